# Initial kernel scaffold; baseline (speedup 1.0000x reference)
import jax
import jax.numpy as jnp
from jax.experimental import pallas as pl


def kernel(boxes, scores):
    raise NotImplementedError("write your pallas kernel here")


# speedup vs baseline: 434.1933x; 434.1933x over previous
import functools

import jax
import jax.numpy as jnp
from jax import lax
from jax.experimental import pallas as pl
from jax.experimental.pallas import tpu as pltpu

IOU_T = 0.7
SCORE_T = 0.5
_N = 20000
_B = 1024
_NB = (_N + _B - 1) // _B
_NP = _NB * _B


def _tile_overlap(colc, rowc):
    x1c, y1c, x2c, y2c, ac = colc
    x1r, y1r, x2r, y2r, ar = rowc
    xx1 = jnp.maximum(x1c, x1r)
    yy1 = jnp.maximum(y1c, y1r)
    xx2 = jnp.minimum(x2c, x2r)
    yy2 = jnp.minimum(y2c, y2r)
    w = jnp.maximum(xx2 - xx1, 0.0)
    h = jnp.maximum(yy2 - yy1, 0.0)
    inter = w * h
    iou = inter / (ac + ar - inter + 1e-9)
    return iou > IOU_T


def _nms_kernel_body(bc_ref, br_ref, sr_ref, keep_ref, supp_ref, *, b, nb):
    keep_ref[...] = jnp.zeros_like(keep_ref)
    supp_ref[...] = jnp.zeros_like(supp_ref)

    m = jnp.sum((sr_ref[...] > SCORE_T).astype(jnp.int32))
    nbv = lax.div(m + (b - 1), b)

    def row_coords(j):
        brj = br_ref[j]
        x1 = brj[0:1, :] - brj[2:3, :] * 0.5
        y1 = brj[1:2, :] - brj[3:4, :] * 0.5
        x2 = brj[0:1, :] + brj[2:3, :] * 0.5
        y2 = brj[1:2, :] + brj[3:4, :] * 0.5
        return (x1, y1, x2, y2, (x2 - x1) * (y2 - y1))

    def col_coords(k):
        bk = bc_ref[pl.ds(k * b, b), :]
        x1 = bk[:, 0:1] - bk[:, 2:3] * 0.5
        y1 = bk[:, 1:2] - bk[:, 3:4] * 0.5
        x2 = bk[:, 0:1] + bk[:, 2:3] * 0.5
        y2 = bk[:, 1:2] + bk[:, 3:4] * 0.5
        return (x1, y1, x2, y2, (x2 - x1) * (y2 - y1))

    def outer(k, carry):
        colc = col_coords(k)
        over_kk = _tile_overlap(colc, row_coords(k))
        ii = lax.broadcasted_iota(jnp.int32, (b, b), 0)
        jj = lax.broadcasted_iota(jnp.int32, (b, b), 1)
        okk = jnp.where(over_kk & (jj > ii), 1.0, 0.0)

        sk = sr_ref[k]
        alive = jnp.where((sk > SCORE_T) & (supp_ref[k] == 0.0), 1.0, 0.0)

        def fp_step(cur):
            s = jnp.dot(cur, okk, preferred_element_type=jnp.float32)
            return alive * jnp.where(s > 0.0, 0.0, 1.0)

        def fp_cond(c):
            prev, cur = c
            return jnp.any(prev != cur)

        def fp_body(c):
            _, cur = c
            return (cur, fp_step(cur))

        _, keep = lax.while_loop(fp_cond, fp_body, (alive, fp_step(alive)))
        keep_ref[k] = keep

        def inner(j, carry):
            over = _tile_overlap(colc, row_coords(j))
            o = jnp.where(over, 1.0, 0.0)
            s = jnp.dot(keep, o, preferred_element_type=jnp.float32)
            supp_ref[j] = jnp.maximum(supp_ref[j], jnp.where(s > 0.0, 1.0, 0.0))
            return carry

        return lax.fori_loop(k + 1, nbv, inner, carry)

    lax.fori_loop(0, nbv, outer, 0)


def _nms_sorted(bs_p, ss_p, b, nb):
    br3 = bs_p.T.reshape(4, nb, b).transpose(1, 0, 2)
    sr3 = ss_p.reshape(nb, 1, b)
    keep = pl.pallas_call(
        functools.partial(_nms_kernel_body, b=b, nb=nb),
        out_shape=jax.ShapeDtypeStruct((nb, 1, b), jnp.float32),
        scratch_shapes=[pltpu.VMEM((nb, 1, b), jnp.float32)],
    )(bs_p, br3, sr3)
    return keep.reshape(nb * b)


def kernel(boxes, scores):
    order = jnp.argsort(-scores)
    bs = boxes[order]
    ss = scores[order]
    pad = _NP - _N
    bs_p = jnp.concatenate([bs, jnp.zeros((pad, 4), boxes.dtype)], axis=0)
    ss_p = jnp.concatenate([ss, jnp.full((pad,), -1.0, scores.dtype)], axis=0)
    keep_s = _nms_sorted(bs_p, ss_p, _B, _NB)[:_N]
    keepf = jnp.zeros((_N,), boxes.dtype).at[order].set(keep_s)
    return jnp.concatenate([boxes * keepf[:, None], (scores * keepf)[:, None]],
                           axis=1)
